# baseline (speedup 1.0000x reference)
import jax
import jax.numpy as jnp
from jax import lax
from jax.experimental import pallas as pl
from jax.experimental.pallas import tpu as pltpu
from jax.experimental.pallas import tpu_sc as plsc

T = 8192
D = 768
E = 4
F = 1024
NG = 6
BLK = 256
P = 9472
NBLK = P // BLK
NW = 32
TPW = T // NW
BT = 512

_LO = (0, 0, 0, 1, 1, 2)
_HI = (1, 2, 3, 2, 3, 3)


def _linrout_body(x_ref, wl_ref, bl_ref, wr_ref, h_ref, pid_ref, glo_ref,
                  ghi_ref, cnt_ref):
    x = x_ref[...]
    h = jnp.dot(x, wl_ref[...]) + bl_ref[...][None, :]
    h_ref[...] = h
    logits = jnp.dot(h, wr_ref[...])
    probs = jax.nn.softmax(logits, axis=-1)
    iota = lax.broadcasted_iota(jnp.int32, (BT, E), 1)
    m0 = jnp.max(probs, axis=-1, keepdims=True)
    i0 = jnp.min(jnp.where(probs == m0, iota, E), axis=-1, keepdims=True)
    probs1 = jnp.where(iota == i0, -1.0, probs)
    m1 = jnp.max(probs1, axis=-1, keepdims=True)
    i1 = jnp.min(jnp.where(probs1 == m1, iota, E), axis=-1, keepdims=True)
    den = m0 + m1
    w0 = m0 / den
    w1 = m1 / den
    lo = jnp.minimum(i0, i1)
    hi = jnp.maximum(i0, i1)
    pid = 3 * lo - (lo * (lo - 1)) // 2 + (hi - lo - 1)
    pid_ref[...] = pid
    glo_ref[...] = jnp.where(lo == i0, w0, w1)
    ghi_ref[...] = jnp.where(lo == i0, w1, w0)
    ohg = (pid == lax.broadcasted_iota(jnp.int32, (BT, NG), 1))
    ohi = jnp.where(ohg, jnp.int32(1), jnp.int32(0))
    cnt_ref[0, 0:1, :] = jnp.sum(ohi[:TPW], axis=0, keepdims=True)
    cnt_ref[0, 1:2, :] = jnp.sum(ohi[TPW:], axis=0, keepdims=True)


def _linrout(x, W_lin, b_lin, W_router):
    return pl.pallas_call(
        _linrout_body,
        grid=(T // BT,),
        in_specs=[
            pl.BlockSpec((BT, D), lambda i: (i, 0)),
            pl.BlockSpec((D, D), lambda i: (0, 0)),
            pl.BlockSpec((D,), lambda i: (0,)),
            pl.BlockSpec((D, E), lambda i: (0, 0)),
        ],
        out_specs=[
            pl.BlockSpec((BT, D), lambda i: (i, 0)),
            pl.BlockSpec((BT, 1), lambda i: (i, 0)),
            pl.BlockSpec((BT, 1), lambda i: (i, 0)),
            pl.BlockSpec((BT, 1), lambda i: (i, 0)),
            pl.BlockSpec((1, 2, NG), lambda i: (i, 0, 0)),
        ],
        out_shape=[
            jax.ShapeDtypeStruct((T, D), jnp.float32),
            jax.ShapeDtypeStruct((T, 1), jnp.int32),
            jax.ShapeDtypeStruct((T, 1), jnp.float32),
            jax.ShapeDtypeStruct((T, 1), jnp.float32),
            jax.ShapeDtypeStruct((T // BT, 2, NG), jnp.int32),
        ],
        compiler_params=pltpu.CompilerParams(
            dimension_semantics=("arbitrary",),
        ),
        name="linrout_tc",
    )(x, W_lin, b_lin, W_router)


def _route_body(pid_hbm, glo_hbm, ghi_hbm, cnts_hbm,
                gpk_hbm, inv_hbm, eab_hbm,
                pid_v, cnts_v, dst2, g8, ga2, gb2, eabv,
                sem, sem_pid, sem_cnt):
    c = lax.axis_index("c")
    s = lax.axis_index("s")
    w = s * 2 + c
    base_tok = w * TPW
    d_pid = pltpu.async_copy(pid_hbm.at[pl.ds(base_tok, TPW)], pid_v, sem_pid)
    d_cnt = pltpu.async_copy(cnts_hbm, cnts_v, sem_cnt)
    d_g = [pltpu.async_copy(glo_hbm.at[pl.ds(base_tok, 128)], ga2.at[0], sem),
           pltpu.async_copy(glo_hbm.at[pl.ds(base_tok + 128, 128)], ga2.at[1],
                            sem),
           pltpu.async_copy(ghi_hbm.at[pl.ds(base_tok, 128)], gb2.at[0], sem),
           pltpu.async_copy(ghi_hbm.at[pl.ds(base_tok + 128, 128)], gb2.at[1],
                            sem)]
    d_cnt.wait()

    vecs = [cnts_v[pl.ds(16 * k, 16)] for k in range(12)]
    cnt = []
    pre = []
    for g in range(NG):
        acc_t = jnp.zeros((16,), jnp.int32)
        acc_p = jnp.zeros((16,), jnp.int32)
        for k in range(12):
            flat = 16 * k + lax.iota(jnp.int32, 16)
            chunk = (flat * 10923) >> 16
            sel = flat - chunk * NG == g
            v = jnp.where(sel, vecs[k], 0)
            acc_t = acc_t + v
            acc_p = acc_p + jnp.where(chunk < w, v, 0)
        cnt.append(jnp.sum(acc_t))
        pre.append(jnp.sum(acc_p))
    seg = [((cg + (BLK - 1)) >> 8) << 8 for cg in cnt]
    base = [jnp.int32(0)]
    for g in range(NG):
        base.append(base[g] + seg[g])
    d_pid.wait()
    run = [jnp.int32(0)] * NG
    for i in range(16):
        v = pid_v[pl.ds(i * 16, 16)]
        dst = jnp.zeros((16,), jnp.int32)
        for g in range(NG):
            m = v == g
            mi = jnp.where(m, jnp.int32(1), jnp.int32(0))
            csum = plsc.cumsum(mi)
            dst = jnp.where(m, base[g] + pre[g] + run[g] + csum - 1, dst)
            run[g] = run[g] + jnp.sum(mi)
        dst = jnp.clip(dst, 0, P - 1)
        j, k = divmod(i, 8)
        dst2[j, pl.ds(k * 16, 16)] = dst

    for d in d_g:
        d.wait()
    ii16 = lax.iota(jnp.int32, 16)
    zc = jnp.zeros((16,), jnp.int32)
    for i in range(16):
        j, k = divmod(i, 8)
        rows = i * 16 + ii16
        plsc.store_scatter(g8, [rows, zc], ga2[j, pl.ds(k * 16, 16)])
        plsc.store_scatter(g8, [rows, zc + 1], gb2[j, pl.ds(k * 16, 16)])
    outs = []
    for j in range(2):
        outs.append(pltpu.async_copy(g8.at[pl.ds(j * 128, 128)],
                                     gpk_hbm.at[dst2.at[j]], sem))
    outs.append(pltpu.async_copy(dst2, inv_hbm.at[pl.ds(2 * w, 2)], sem))

    @pl.when(w == 0)
    def _():
        for i in range(4):
            pos = (i * 16 + lax.iota(jnp.int32, 16)) * BLK
            gid = jnp.zeros((16,), jnp.int32)
            for g in range(1, NG):
                gid = gid + jnp.where(pos >= base[g], jnp.int32(1),
                                      jnp.int32(0))
            ea = jnp.zeros((16,), jnp.int32)
            eb = jnp.zeros((16,), jnp.int32)
            for g in range(NG):
                ea = jnp.where(gid == g, jnp.int32(_LO[g]), ea)
                eb = jnp.where(gid == g, jnp.int32(_HI[g]), eb)
            eabv[i, :] = ea
            eabv[4 + i, :] = eb
        pltpu.async_copy(eabv, eab_hbm, sem).wait()

    for d in outs:
        d.wait()


def _route(pid, glo, ghi, cnts):
    mesh = plsc.VectorSubcoreMesh(core_axis_name="c", subcore_axis_name="s")
    f = pl.kernel(
        _route_body,
        out_type=(
            jax.ShapeDtypeStruct((P, 128), jnp.float32),
            jax.ShapeDtypeStruct((NW * 2, 128), jnp.int32),
            jax.ShapeDtypeStruct((8, 16), jnp.int32),
        ),
        mesh=mesh,
        scratch_types=[
            pltpu.VMEM((TPW,), jnp.int32),
            pltpu.VMEM((NW * NG,), jnp.int32),
            pltpu.VMEM((2, 128), jnp.int32),
            pltpu.VMEM((TPW, 128), jnp.float32),
            pltpu.VMEM((2, 128), jnp.float32),
            pltpu.VMEM((2, 128), jnp.float32),
            pltpu.VMEM((8, 16), jnp.int32),
            pltpu.SemaphoreType.DMA,
            pltpu.SemaphoreType.DMA,
            pltpu.SemaphoreType.DMA,
        ],
        compiler_params=pltpu.CompilerParams(needs_layout_passes=False),
        name="route_sc",
    )
    return f(pid, glo, ghi, cnts)


def _hscatter_body(h_hbm, inv_hbm, hp_hbm, idx2, rows0, rows1, gsem, wsem0,
                   wsem1):
    c = lax.axis_index("c")
    s = lax.axis_index("s")
    w = s * 2 + c
    r0 = w * 256
    for i in range(4):
        pltpu.sync_copy(inv_hbm.at[pl.ds(r0 + i * 64, 64)], idx2.at[i])
    for i in range(4):
        for k in range(4):
            idx2[i, pl.ds(k * 16, 16)] = jnp.clip(
                idx2[i, pl.ds(k * 16, 16)], 0, P - 1)
    bufs = (rows0, rows1)
    wsems = (wsem0, wsem1)
    wds = []
    for i in range(4):
        if i >= 2:
            wds[i - 2].wait()
        g = pltpu.async_copy(h_hbm.at[pl.ds(r0 + i * 64, 64)], bufs[i % 2],
                             gsem)
        g.wait()
        wds.append(pltpu.async_copy(bufs[i % 2], hp_hbm.at[idx2.at[i]],
                                    wsems[i % 2]))
    wds[2].wait()
    wds[3].wait()


def _hscatter(h_f32, inv):
    mesh = plsc.VectorSubcoreMesh(core_axis_name="c", subcore_axis_name="s")
    f = pl.kernel(
        _hscatter_body,
        out_type=jax.ShapeDtypeStruct((P, D), jnp.float32),
        mesh=mesh,
        scratch_types=[
            pltpu.VMEM((4, 64), jnp.int32),
            pltpu.VMEM((64, D), jnp.float32),
            pltpu.VMEM((64, D), jnp.float32),
            pltpu.SemaphoreType.DMA,
            pltpu.SemaphoreType.DMA,
            pltpu.SemaphoreType.DMA,
        ],
        compiler_params=pltpu.CompilerParams(needs_layout_passes=False),
        name="hscatter_sc",
    )
    return f(h_f32, inv)


def _ffn_body(ea_ref, eb_ref, hp_ref, gpk_ref,
              w1a_ref, b1a_ref, w2a_ref, b2a_ref,
              w1b_ref, b1b_ref, w2b_ref, b2b_ref, yp_ref):
    ga_ref = gpk_ref[:, 0:1]
    gb_ref = gpk_ref[:, 1:2]
    hp = hp_ref[...].astype(jnp.bfloat16)
    a1 = jnp.dot(hp, w1a_ref[0], preferred_element_type=jnp.float32)
    a1 = jnp.maximum(a1 + b1a_ref[0], 0.0).astype(jnp.bfloat16)
    ya = jnp.dot(a1, w2a_ref[0], preferred_element_type=jnp.float32)
    ya = ya + b2a_ref[0]
    b1v = jnp.dot(hp, w1b_ref[0], preferred_element_type=jnp.float32)
    b1v = jnp.maximum(b1v + b1b_ref[0], 0.0).astype(jnp.bfloat16)
    yb = jnp.dot(b1v, w2b_ref[0], preferred_element_type=jnp.float32)
    yb = yb + b2b_ref[0]
    yp_ref[...] = ga_ref * ya + gb_ref * yb


def _ffn(ea, eb, hp_bf, gpk, W1b, b1, W2b, b2):
    grid_spec = pltpu.PrefetchScalarGridSpec(
        num_scalar_prefetch=2,
        grid=(NBLK,),
        in_specs=[
            pl.BlockSpec((BLK, D), lambda i, ea, eb: (i, 0)),
            pl.BlockSpec((BLK, 128), lambda i, ea, eb: (i, 0)),
            pl.BlockSpec((1, D, F), lambda i, ea, eb: (ea[i], 0, 0)),
            pl.BlockSpec((1, 1, F), lambda i, ea, eb: (ea[i], 0, 0)),
            pl.BlockSpec((1, F, D), lambda i, ea, eb: (ea[i], 0, 0)),
            pl.BlockSpec((1, 1, D), lambda i, ea, eb: (ea[i], 0, 0)),
            pl.BlockSpec((1, D, F), lambda i, ea, eb: (eb[i], 0, 0)),
            pl.BlockSpec((1, 1, F), lambda i, ea, eb: (eb[i], 0, 0)),
            pl.BlockSpec((1, F, D), lambda i, ea, eb: (eb[i], 0, 0)),
            pl.BlockSpec((1, 1, D), lambda i, ea, eb: (eb[i], 0, 0)),
        ],
        out_specs=pl.BlockSpec((BLK, D), lambda i, ea, eb: (i, 0)),
    )
    return pl.pallas_call(
        _ffn_body,
        grid_spec=grid_spec,
        out_shape=jax.ShapeDtypeStruct((P, D), jnp.float32),
        compiler_params=pltpu.CompilerParams(
            dimension_semantics=("arbitrary",),
        ),
        name="ffn_tc",
    )(ea, eb, hp_bf, gpk, W1b, b1, W2b, b2, W1b, b1, W2b, b2)


def _fin_body(yp_hbm, inv_hbm, out_hbm, idx_v, rows0, rows1, gsem, wsem0,
              wsem1):
    c = lax.axis_index("c")
    s = lax.axis_index("s")
    w = s * 2 + c
    r0 = w * 256
    pltpu.sync_copy(inv_hbm.at[pl.ds(r0, 256)], idx_v)
    for i in range(16):
        idx_v[pl.ds(i * 16, 16)] = jnp.clip(idx_v[pl.ds(i * 16, 16)], 0, P - 1)
    bufs = (rows0, rows1)
    wsems = (wsem0, wsem1)
    wds = []
    for i in range(4):
        if i >= 2:
            wds[i - 2].wait()
        g = pltpu.async_copy(yp_hbm.at[idx_v.at[pl.ds(i * 64, 64)]],
                             bufs[i % 2], gsem)
        g.wait()
        wds.append(pltpu.async_copy(bufs[i % 2],
                                    out_hbm.at[pl.ds(r0 + i * 64, 64)],
                                    wsems[i % 2]))
    wds[2].wait()
    wds[3].wait()


def _fin(yp, inv):
    mesh = plsc.VectorSubcoreMesh(core_axis_name="c", subcore_axis_name="s")
    f = pl.kernel(
        _fin_body,
        out_type=jax.ShapeDtypeStruct((T, D), jnp.float32),
        mesh=mesh,
        scratch_types=[
            pltpu.VMEM((256,), jnp.int32),
            pltpu.VMEM((64, D), jnp.float32),
            pltpu.VMEM((64, D), jnp.float32),
            pltpu.SemaphoreType.DMA,
            pltpu.SemaphoreType.DMA,
            pltpu.SemaphoreType.DMA,
        ],
        compiler_params=pltpu.CompilerParams(needs_layout_passes=False),
        name="fingather_sc",
    )
    return f(yp, inv)


def kernel(input_ids, W_lin, b_lin, W_router, W1, b1, W2, b2):
    B, S, _ = input_ids.shape
    x = input_ids.reshape(T, D)
    h_f32, pid2, glo2, ghi2, cnts3 = _linrout(x, W_lin, b_lin, W_router)
    pid = pid2.reshape(T)
    glo = glo2.reshape(T)
    ghi = ghi2.reshape(T)
    gpk, inv2, eab = _route(pid, glo, ghi, cnts3.reshape(NW * NG))
    inv = inv2.reshape(T)
    hp = _hscatter(h_f32, inv)
    eabf = eab.reshape(128)
    ea = eabf[:NBLK]
    eb = eabf[64:64 + NBLK]
    yp = _ffn(ea, eb, hp, gpk,
              W1.astype(jnp.bfloat16), b1.reshape(E, 1, F),
              W2.astype(jnp.bfloat16), b2.reshape(E, 1, D))
    out = _fin(yp, inv)
    return out.reshape(B, S, D)


# speedup vs baseline: 1.0142x; 1.0142x over previous
import jax
import jax.numpy as jnp
from jax import lax
from jax.experimental import pallas as pl
from jax.experimental.pallas import tpu as pltpu
from jax.experimental.pallas import tpu_sc as plsc

T = 8192
D = 768
E = 4
F = 1024
NG = 6
BLK = 256
P = 9472
NBLK = P // BLK
NW = 32
TPW = T // NW
BT = 512

_LO = (0, 0, 0, 1, 1, 2)
_HI = (1, 2, 3, 2, 3, 3)


def _linrout_body(x_ref, wl_ref, bl_ref, wr_ref, h_ref, pid_ref, glo_ref,
                  ghi_ref, cnt_ref):
    x = x_ref[...]
    h = jnp.dot(x, wl_ref[...]) + bl_ref[...][None, :]
    h_ref[...] = h
    logits = jnp.dot(h, wr_ref[...])
    probs = jax.nn.softmax(logits, axis=-1)
    iota = lax.broadcasted_iota(jnp.int32, (BT, E), 1)
    m0 = jnp.max(probs, axis=-1, keepdims=True)
    i0 = jnp.min(jnp.where(probs == m0, iota, E), axis=-1, keepdims=True)
    probs1 = jnp.where(iota == i0, -1.0, probs)
    m1 = jnp.max(probs1, axis=-1, keepdims=True)
    i1 = jnp.min(jnp.where(probs1 == m1, iota, E), axis=-1, keepdims=True)
    den = m0 + m1
    w0 = m0 / den
    w1 = m1 / den
    lo = jnp.minimum(i0, i1)
    hi = jnp.maximum(i0, i1)
    pid = 3 * lo - (lo * (lo - 1)) // 2 + (hi - lo - 1)
    pid_ref[...] = pid
    glo_ref[...] = jnp.where(lo == i0, w0, w1)
    ghi_ref[...] = jnp.where(lo == i0, w1, w0)
    ohg = (pid == lax.broadcasted_iota(jnp.int32, (BT, NG), 1))
    ohi = jnp.where(ohg, jnp.int32(1), jnp.int32(0))
    cnt_ref[0, 0:1, :] = jnp.sum(ohi[:TPW], axis=0, keepdims=True)
    cnt_ref[0, 1:2, :] = jnp.sum(ohi[TPW:], axis=0, keepdims=True)


def _linrout(x, W_lin, b_lin, W_router):
    return pl.pallas_call(
        _linrout_body,
        grid=(T // BT,),
        in_specs=[
            pl.BlockSpec((BT, D), lambda i: (i, 0)),
            pl.BlockSpec((D, D), lambda i: (0, 0)),
            pl.BlockSpec((D,), lambda i: (0,)),
            pl.BlockSpec((D, E), lambda i: (0, 0)),
        ],
        out_specs=[
            pl.BlockSpec((BT, D), lambda i: (i, 0)),
            pl.BlockSpec((BT, 1), lambda i: (i, 0)),
            pl.BlockSpec((BT, 1), lambda i: (i, 0)),
            pl.BlockSpec((BT, 1), lambda i: (i, 0)),
            pl.BlockSpec((1, 2, NG), lambda i: (i, 0, 0)),
        ],
        out_shape=[
            jax.ShapeDtypeStruct((T, D), jnp.float32),
            jax.ShapeDtypeStruct((T, 1), jnp.int32),
            jax.ShapeDtypeStruct((T, 1), jnp.float32),
            jax.ShapeDtypeStruct((T, 1), jnp.float32),
            jax.ShapeDtypeStruct((T // BT, 2, NG), jnp.int32),
        ],
        compiler_params=pltpu.CompilerParams(
            dimension_semantics=("arbitrary",),
        ),
        name="linrout_tc",
    )(x, W_lin, b_lin, W_router)


def _route_body(pid_hbm, glo_hbm, ghi_hbm, cnts_hbm, h_hbm,
                gpk_hbm, inv_hbm, eab_hbm, hp_hbm,
                pid_v, cnts_v, dst2, idx4, g8, ga2, gb2, eabv, rows0, rows1,
                sem, sem_pid, sem_cnt, gsem, wsem0, wsem1):
    c = lax.axis_index("c")
    s = lax.axis_index("s")
    w = s * 2 + c
    base_tok = w * TPW
    d_pid = pltpu.async_copy(pid_hbm.at[pl.ds(base_tok, TPW)], pid_v, sem_pid)
    d_cnt = pltpu.async_copy(cnts_hbm, cnts_v, sem_cnt)
    d_g = [pltpu.async_copy(glo_hbm.at[pl.ds(base_tok, 128)], ga2.at[0], sem),
           pltpu.async_copy(glo_hbm.at[pl.ds(base_tok + 128, 128)], ga2.at[1],
                            sem),
           pltpu.async_copy(ghi_hbm.at[pl.ds(base_tok, 128)], gb2.at[0], sem),
           pltpu.async_copy(ghi_hbm.at[pl.ds(base_tok + 128, 128)], gb2.at[1],
                            sem)]
    d_cnt.wait()

    vecs = [cnts_v[pl.ds(16 * k, 16)] for k in range(12)]
    cnt = []
    pre = []
    for g in range(NG):
        acc_t = jnp.zeros((16,), jnp.int32)
        acc_p = jnp.zeros((16,), jnp.int32)
        for k in range(12):
            flat = 16 * k + lax.iota(jnp.int32, 16)
            chunk = (flat * 10923) >> 16
            sel = flat - chunk * NG == g
            v = jnp.where(sel, vecs[k], 0)
            acc_t = acc_t + v
            acc_p = acc_p + jnp.where(chunk < w, v, 0)
        cnt.append(jnp.sum(acc_t))
        pre.append(jnp.sum(acc_p))
    seg = [((cg + (BLK - 1)) >> 8) << 8 for cg in cnt]
    base = [jnp.int32(0)]
    for g in range(NG):
        base.append(base[g] + seg[g])
    d_pid.wait()
    run = [jnp.int32(0)] * NG
    for i in range(16):
        v = pid_v[pl.ds(i * 16, 16)]
        dst = jnp.zeros((16,), jnp.int32)
        for g in range(NG):
            m = v == g
            mi = jnp.where(m, jnp.int32(1), jnp.int32(0))
            csum = plsc.cumsum(mi)
            dst = jnp.where(m, base[g] + pre[g] + run[g] + csum - 1, dst)
            run[g] = run[g] + jnp.sum(mi)
        dst = jnp.clip(dst, 0, P - 1)
        j, k = divmod(i, 8)
        dst2[j, pl.ds(k * 16, 16)] = dst
        idx4[i // 2, pl.ds((i % 2) * 16, 16)] = dst

    for d in d_g:
        d.wait()
    ii16 = lax.iota(jnp.int32, 16)
    zc = jnp.zeros((16,), jnp.int32)
    for i in range(16):
        j, k = divmod(i, 8)
        rows = i * 16 + ii16
        plsc.store_scatter(g8, [rows, zc], ga2[j, pl.ds(k * 16, 16)])
        plsc.store_scatter(g8, [rows, zc + 1], gb2[j, pl.ds(k * 16, 16)])
    outs = []
    for j in range(2):
        outs.append(pltpu.async_copy(g8.at[pl.ds(j * 128, 128)],
                                     gpk_hbm.at[dst2.at[j]], sem))
    outs.append(pltpu.async_copy(dst2, inv_hbm.at[pl.ds(2 * w, 2)], sem))

    r0 = w * TPW
    bufs = (rows0, rows1)
    wsems = (wsem0, wsem1)
    wds = []
    for i in range(8):
        if i >= 2:
            wds[i - 2].wait()
        gld = pltpu.async_copy(h_hbm.at[pl.ds(r0 + i * 32, 32)], bufs[i % 2],
                               gsem)
        gld.wait()
        wds.append(pltpu.async_copy(bufs[i % 2], hp_hbm.at[idx4.at[i]],
                                    wsems[i % 2]))
    wds[6].wait()
    wds[7].wait()

    @pl.when(w == 0)
    def _():
        for i in range(4):
            pos = (i * 16 + lax.iota(jnp.int32, 16)) * BLK
            gid = jnp.zeros((16,), jnp.int32)
            for g in range(1, NG):
                gid = gid + jnp.where(pos >= base[g], jnp.int32(1),
                                      jnp.int32(0))
            ea = jnp.zeros((16,), jnp.int32)
            eb = jnp.zeros((16,), jnp.int32)
            for g in range(NG):
                ea = jnp.where(gid == g, jnp.int32(_LO[g]), ea)
                eb = jnp.where(gid == g, jnp.int32(_HI[g]), eb)
            eabv[i, :] = ea
            eabv[4 + i, :] = eb
        pltpu.async_copy(eabv, eab_hbm, sem).wait()

    for d in outs:
        d.wait()


def _route(pid, glo, ghi, cnts, h_f32):
    mesh = plsc.VectorSubcoreMesh(core_axis_name="c", subcore_axis_name="s")
    f = pl.kernel(
        _route_body,
        out_type=(
            jax.ShapeDtypeStruct((P, 128), jnp.float32),
            jax.ShapeDtypeStruct((NW * 2, 128), jnp.int32),
            jax.ShapeDtypeStruct((8, 16), jnp.int32),
            jax.ShapeDtypeStruct((P, D), jnp.float32),
        ),
        mesh=mesh,
        scratch_types=[
            pltpu.VMEM((TPW,), jnp.int32),
            pltpu.VMEM((NW * NG,), jnp.int32),
            pltpu.VMEM((2, 128), jnp.int32),
            pltpu.VMEM((8, 32), jnp.int32),
            pltpu.VMEM((TPW, 128), jnp.float32),
            pltpu.VMEM((2, 128), jnp.float32),
            pltpu.VMEM((2, 128), jnp.float32),
            pltpu.VMEM((8, 16), jnp.int32),
            pltpu.VMEM((32, D), jnp.float32),
            pltpu.VMEM((32, D), jnp.float32),
            pltpu.SemaphoreType.DMA,
            pltpu.SemaphoreType.DMA,
            pltpu.SemaphoreType.DMA,
            pltpu.SemaphoreType.DMA,
            pltpu.SemaphoreType.DMA,
            pltpu.SemaphoreType.DMA,
        ],
        compiler_params=pltpu.CompilerParams(needs_layout_passes=False),
        name="route_sc",
    )
    return f(pid, glo, ghi, cnts, h_f32)


def _ffn_body(ea_ref, eb_ref, hp_ref, gpk_ref,
              w1a_ref, b1a_ref, w2a_ref, b2a_ref,
              w1b_ref, b1b_ref, w2b_ref, b2b_ref, yp_ref):
    ga_ref = gpk_ref[:, 0:1]
    gb_ref = gpk_ref[:, 1:2]
    hp = hp_ref[...].astype(jnp.bfloat16)
    a1 = jnp.dot(hp, w1a_ref[0], preferred_element_type=jnp.float32)
    a1 = jnp.maximum(a1 + b1a_ref[0], 0.0).astype(jnp.bfloat16)
    ya = jnp.dot(a1, w2a_ref[0], preferred_element_type=jnp.float32)
    ya = ya + b2a_ref[0]
    b1v = jnp.dot(hp, w1b_ref[0], preferred_element_type=jnp.float32)
    b1v = jnp.maximum(b1v + b1b_ref[0], 0.0).astype(jnp.bfloat16)
    yb = jnp.dot(b1v, w2b_ref[0], preferred_element_type=jnp.float32)
    yb = yb + b2b_ref[0]
    yp_ref[...] = ga_ref * ya + gb_ref * yb


def _ffn(ea, eb, hp_bf, gpk, W1b, b1, W2b, b2):
    grid_spec = pltpu.PrefetchScalarGridSpec(
        num_scalar_prefetch=2,
        grid=(NBLK,),
        in_specs=[
            pl.BlockSpec((BLK, D), lambda i, ea, eb: (i, 0)),
            pl.BlockSpec((BLK, 128), lambda i, ea, eb: (i, 0)),
            pl.BlockSpec((1, D, F), lambda i, ea, eb: (ea[i], 0, 0)),
            pl.BlockSpec((1, 1, F), lambda i, ea, eb: (ea[i], 0, 0)),
            pl.BlockSpec((1, F, D), lambda i, ea, eb: (ea[i], 0, 0)),
            pl.BlockSpec((1, 1, D), lambda i, ea, eb: (ea[i], 0, 0)),
            pl.BlockSpec((1, D, F), lambda i, ea, eb: (eb[i], 0, 0)),
            pl.BlockSpec((1, 1, F), lambda i, ea, eb: (eb[i], 0, 0)),
            pl.BlockSpec((1, F, D), lambda i, ea, eb: (eb[i], 0, 0)),
            pl.BlockSpec((1, 1, D), lambda i, ea, eb: (eb[i], 0, 0)),
        ],
        out_specs=pl.BlockSpec((BLK, D), lambda i, ea, eb: (i, 0)),
    )
    return pl.pallas_call(
        _ffn_body,
        grid_spec=grid_spec,
        out_shape=jax.ShapeDtypeStruct((P, D), jnp.float32),
        compiler_params=pltpu.CompilerParams(
            dimension_semantics=("arbitrary",),
        ),
        name="ffn_tc",
    )(ea, eb, hp_bf, gpk, W1b, b1, W2b, b2, W1b, b1, W2b, b2)


def _fin_body(yp_hbm, inv_hbm, out_hbm, idx_v, rows0, rows1, gsem, wsem0,
              wsem1):
    c = lax.axis_index("c")
    s = lax.axis_index("s")
    w = s * 2 + c
    r0 = w * 256
    pltpu.sync_copy(inv_hbm.at[pl.ds(r0, 256)], idx_v)
    for i in range(16):
        idx_v[pl.ds(i * 16, 16)] = jnp.clip(idx_v[pl.ds(i * 16, 16)], 0, P - 1)
    bufs = (rows0, rows1)
    wsems = (wsem0, wsem1)
    wds = []
    for i in range(4):
        if i >= 2:
            wds[i - 2].wait()
        g = pltpu.async_copy(yp_hbm.at[idx_v.at[pl.ds(i * 64, 64)]],
                             bufs[i % 2], gsem)
        g.wait()
        wds.append(pltpu.async_copy(bufs[i % 2],
                                    out_hbm.at[pl.ds(r0 + i * 64, 64)],
                                    wsems[i % 2]))
    wds[2].wait()
    wds[3].wait()


def _fin(yp, inv):
    mesh = plsc.VectorSubcoreMesh(core_axis_name="c", subcore_axis_name="s")
    f = pl.kernel(
        _fin_body,
        out_type=jax.ShapeDtypeStruct((T, D), jnp.float32),
        mesh=mesh,
        scratch_types=[
            pltpu.VMEM((256,), jnp.int32),
            pltpu.VMEM((64, D), jnp.float32),
            pltpu.VMEM((64, D), jnp.float32),
            pltpu.SemaphoreType.DMA,
            pltpu.SemaphoreType.DMA,
            pltpu.SemaphoreType.DMA,
        ],
        compiler_params=pltpu.CompilerParams(needs_layout_passes=False),
        name="fingather_sc",
    )
    return f(yp, inv)


def kernel(input_ids, W_lin, b_lin, W_router, W1, b1, W2, b2):
    B, S, _ = input_ids.shape
    x = input_ids.reshape(T, D)
    h_f32, pid2, glo2, ghi2, cnts3 = _linrout(x, W_lin, b_lin, W_router)
    pid = pid2.reshape(T)
    glo = glo2.reshape(T)
    ghi = ghi2.reshape(T)
    gpk, inv2, eab, hp = _route(pid, glo, ghi, cnts3.reshape(NW * NG), h_f32)
    inv = inv2.reshape(T)
    eabf = eab.reshape(128)
    ea = eabf[:NBLK]
    eb = eabf[64:64 + NBLK]
    yp = _ffn(ea, eb, hp, gpk,
              W1.astype(jnp.bfloat16), b1.reshape(E, 1, F),
              W2.astype(jnp.bfloat16), b2.reshape(E, 1, D))
    out = _fin(yp, inv)
    return out.reshape(B, S, D)
